# baseline (speedup 1.0000x reference)
import functools

import jax
import jax.numpy as jnp
from jax import lax
from jax.experimental import pallas as pl
from jax.experimental.pallas import tpu as pltpu
from jax.experimental.pallas import tpu_sc as plsc

f32 = jnp.float32
i32 = jnp.int32

N = 10000
E = 320000
NRELS = 16
HID = 128
NBK = 16
BN = 640
LR = BN * NRELS
ACC_R = 10256
STRIPE = ACC_R // 16
_ZCH = (82, 82, 82, 82, 82, 82, 82, 67)
GR = NBK * LR
CAPP = E + 520
NC, NS = 2, 16
NW = NC * NS
EW = E // NW
BT = 128

_mesh = plsc.VectorSubcoreMesh(core_axis_name="c", subcore_axis_name="s")


def _wid():
    return lax.axis_index("s") * NC + lax.axis_index("c")


def _lane():
    return lax.iota(i32, 16)


def _i16(v):
    return jnp.full((16,), v, i32)


_ONE16 = None


def _div640(d):
    return ((d >> 7) * 205) >> 10


def _count16(m):
    return jnp.sum(jnp.where(m, _i16(1), _i16(0)))


def _count_body(dst_hbm, cnt_hbm, dvm, cvm):
    wid = _wid()
    lane = _lane()
    pltpu.sync_copy(dst_hbm.at[pl.ds(wid * EW, EW)], dvm)

    def step(j, accs):
        d = dvm[pl.ds(j * 16, 16)]
        bk = _div640(d)
        return tuple(
            accs[b] + jnp.where(bk == _i16(b), _i16(1), _i16(0))
            for b in range(NBK)
        )

    accs = lax.fori_loop(0, EW // 16, step,
                         tuple(jnp.zeros((16,), i32) for _ in range(NBK)))
    cnts = jnp.zeros((16,), i32)
    for b in range(NBK):
        cnts = jnp.where(lane == _i16(b), _i16(jnp.sum(accs[b])), cnts)
    cvm[...] = cnts
    pltpu.sync_copy(cvm, cnt_hbm.at[pl.ds(wid * 16, 16)])


@functools.partial(
    pl.kernel,
    out_type=jax.ShapeDtypeStruct((NW * 16,), i32),
    mesh=_mesh,
    compiler_params=pltpu.CompilerParams(needs_layout_passes=False),
    scratch_types=[pltpu.VMEM((EW,), i32), pltpu.VMEM((16,), i32)],
    name="rgcn_bucket_count",
)
def _k_count(dst_hbm, cnt_hbm, dvm, cvm):
    _count_body(dst_hbm, cnt_hbm, dvm, cvm)


GV = 5


def _bucket_body(src_hbm, dst_hbm, rel_hbm, cnt_hbm, srcb_hbm, lidxb_hbm,
                 svm, dvm, rvm, cvm, offv, ssrc, slidx, spos):
    wid = _wid()
    lane = _lane()
    pltpu.sync_copy(src_hbm.at[pl.ds(wid * EW, EW)], svm)
    pltpu.sync_copy(dst_hbm.at[pl.ds(wid * EW, EW)], dvm)
    pltpu.sync_copy(rel_hbm.at[pl.ds(wid * EW, EW)], rvm)
    pltpu.sync_copy(cnt_hbm, cvm)

    off = lane * CAPP

    def prior(w, off):
        row = cvm[pl.ds(w * 16, 16)]
        keep = _i16(w) < _i16(wid)
        return off + jnp.where(keep, row, _i16(0))

    off = lax.fori_loop(0, NW, prior, off)
    offv[...] = off

    def group(g, _):
        for kk in range(GV):
            jj = g * GV + kk
            d = dvm[pl.ds(jj * 16, 16)]
            s_ = svm[pl.ds(jj * 16, 16)]
            r_ = rvm[pl.ds(jj * 16, 16)]
            bk = _div640(d)
            loc = (d - bk * BN) * NRELS + r_
            rank = jnp.zeros((16,), i32)
            later = jnp.zeros((16,), i32)
            for sh in range(1, 16):
                pv = bk.at[(lane - _i16(sh)) & _i16(15)].get(
                    mode="promise_in_bounds")
                nx = bk.at[(lane + _i16(sh)) & _i16(15)].get(
                    mode="promise_in_bounds")
                rank = rank + jnp.where(
                    (lane >= _i16(sh)) & (pv == bk), _i16(1), _i16(0))
                later = later | jnp.where(
                    (lane + _i16(sh) < _i16(16)) & (nx == bk),
                    _i16(1), _i16(0))
            pos = plsc.load_gather(offv, [bk]) + rank
            plsc.store_scatter(offv, [bk], pos + _i16(1),
                               mask=later == _i16(0))
            ssrc[pl.ds(kk * 16, 16)] = s_
            slidx[pl.ds(kk * 16, 16)] = loc
            spos[pl.ds(kk * 16, 16)] = pos
        pltpu.sync_copy(ssrc, srcb_hbm.at[spos])
        pltpu.sync_copy(slidx, lidxb_hbm.at[spos])
        return 0

    lax.fori_loop(0, EW // (16 * GV), group, 0)


@functools.partial(
    pl.kernel,
    out_type=(jax.ShapeDtypeStruct((NBK * CAPP,), i32),
              jax.ShapeDtypeStruct((NBK * CAPP,), i32)),
    mesh=_mesh,
    compiler_params=pltpu.CompilerParams(needs_layout_passes=False),
    scratch_types=[
        pltpu.VMEM((EW,), i32), pltpu.VMEM((EW,), i32), pltpu.VMEM((EW,), i32),
        pltpu.VMEM((NW * 16,), i32), pltpu.VMEM((16,), i32),
        pltpu.VMEM((GV * 16,), i32), pltpu.VMEM((GV * 16,), i32),
        pltpu.VMEM((GV * 16,), i32),
    ],
    name="rgcn_bucket_scatter",
)
def _k_bucket(src_hbm, dst_hbm, rel_hbm, cnt_hbm, srcb_hbm, lidxb_hbm,
              svm, dvm, rvm, cvm, offv, ssrc, slidx, spos):
    _bucket_body(src_hbm, dst_hbm, rel_hbm, cnt_hbm, srcb_hbm, lidxb_hbm,
                 svm, dvm, rvm, cvm, offv, ssrc, slidx, spos)


def _share(totv, b, sw):
    lane = _lane()
    tot = jnp.sum(jnp.where(lane == _i16(b), totv[...], _i16(0)))
    u = (tot + 7) >> 3
    lo = 8 * ((u * sw) >> 4)
    hi = jnp.minimum(8 * ((u * (sw + 1)) >> 4), tot)
    return tot, lo, hi


def _seg_body(h_hbm, srcb_hbm, lidxb_hbm, tot_hbm, acc_hbm,
              totv, sstage, lstage, sA_m, lA_m, sB_m, lB_m, rowsA, rowsB,
              zbuf, acc_sh, semA, semB):
    c = lax.axis_index("c")
    sw = lax.axis_index("s")
    lane = _lane()
    pltpu.sync_copy(tot_hbm, totv)
    zv = jnp.zeros((16,), f32)
    bufs = ((sA_m, lA_m, rowsA, semA), (sB_m, lB_m, rowsB, semB))

    def zfill(i, _):
        for kk in range(8):
            zbuf[i, pl.ds(kk * 16, 16)] = zv
        return 0

    lax.fori_loop(0, 82, zfill, 0)

    for bl in range(NBK // NC):
        b = c * (NBK // NC) + bl
        zo = 0
        for zch in _ZCH:
            pltpu.sync_copy(zbuf.at[pl.ds(0, zch)],
                            acc_sh.at[pl.ds(sw * STRIPE + zo, zch)])
            zo += zch
        plsc.subcore_barrier()
        _, lo, hi = _share(totv, b, sw)
        nbatch = (hi - lo + BT - 1) >> 7

        def prep(it, par):
            base = lo + it * BT
            sm, lm, rw, se = bufs[par]
            pltpu.sync_copy(srcb_hbm.at[pl.ds(b * CAPP + base, BT)], sstage)
            pltpu.sync_copy(lidxb_hbm.at[pl.ds(b * CAPP + base, BT)], lstage)
            for kk in range(8):
                sv = sstage[pl.ds(kk * 16, 16)]
                lv = lstage[pl.ds(kk * 16, 16)]
                valid = (_i16(base + kk * 16) + lane) < _i16(hi)
                sm[pl.ds(kk * 16, 16)] = jnp.where(valid, sv, lane * 577)
                lm[pl.ds(kk * 16, 16)] = jnp.where(valid, lv, LR + lane)
            pltpu.async_copy(h_hbm.at[sm], rw, se)

        @pl.when(nbatch > 0)
        def _():
            prep(0, 0)

        def pipe(i2, _):
            for par in range(2):
                it = i2 * 2 + par

                @pl.when(it + 1 < nbatch)
                def _():
                    prep(it + 1, 1 - par)

                @pl.when(it < nbatch)
                def _():
                    sm, lm, rw, se = bufs[par]
                    pltpu.make_async_copy(h_hbm.at[sm], rw, se).wait()
                    pltpu.sync_copy(rw, acc_sh.at[lm], add=True)
            return 0

        lax.fori_loop(0, (nbatch + 1) >> 1, pipe, 0)
        plsc.subcore_barrier()
        pltpu.sync_copy(acc_sh.at[pl.ds(sw * BN, BN)],
                        acc_hbm.at[pl.ds(b * LR + sw * BN, BN)])
        plsc.subcore_barrier()


@functools.partial(
    pl.kernel,
    out_type=jax.ShapeDtypeStruct((GR, HID), f32),
    mesh=_mesh,
    compiler_params=pltpu.CompilerParams(needs_layout_passes=False),
    scratch_types=[
        pltpu.VMEM((16,), i32),
        pltpu.VMEM((BT,), i32), pltpu.VMEM((BT,), i32),
        pltpu.VMEM((BT,), i32), pltpu.VMEM((BT,), i32),
        pltpu.VMEM((BT,), i32), pltpu.VMEM((BT,), i32),
        pltpu.VMEM((BT, HID), f32), pltpu.VMEM((BT, HID), f32),
        pltpu.VMEM((82, HID), f32),
        pltpu.VMEM_SHARED((ACC_R, HID), f32),
        pltpu.SemaphoreType.DMA, pltpu.SemaphoreType.DMA,
    ],
    name="rgcn_seg_sum",
)
def _k_seg(h_hbm, srcb_hbm, lidxb_hbm, tot_hbm, acc_hbm,
           totv, sstage, lstage, sA_m, lA_m, sB_m, lB_m, rowsA, rowsB,
           zbuf, acc_sh, semA, semB):
    _seg_body(h_hbm, srcb_hbm, lidxb_hbm, tot_hbm, acc_hbm,
              totv, sstage, lstage, sA_m, lA_m, sB_m, lB_m, rowsA, rowsB,
              zbuf, acc_sh, semA, semB)


def _cnt_body(lidxb_hbm, tot_hbm, cnt_hbm,
              totv, lstage, lstage_m, ones_rows, zbuf, fbuf, cnt_sh):
    c = lax.axis_index("c")
    sw = lax.axis_index("s")
    lane = _lane()
    pltpu.sync_copy(tot_hbm, totv)
    zv = jnp.zeros((16,), f32)
    ov = jnp.full((16,), 1.0, f32)

    def zfill(i, _):
        for kk in range(8):
            zbuf[i, pl.ds(kk * 16, 16)] = zv
            ones_rows[i, pl.ds(kk * 16, 16)] = ov
        return 0

    lax.fori_loop(0, 82, zfill, 0)

    def ofill(i, _):
        for kk in range(8):
            ones_rows[82 + i, pl.ds(kk * 16, 16)] = ov
        return 0

    lax.fori_loop(0, BT - 82, ofill, 0)

    for bl in range(NBK // NC):
        b = c * (NBK // NC) + bl
        zo = 0
        for zch in _ZCH:
            pltpu.sync_copy(zbuf.at[pl.ds(0, zch)],
                            cnt_sh.at[pl.ds(sw * STRIPE + zo, zch)])
            zo += zch
        plsc.subcore_barrier()
        _, lo, hi = _share(totv, b, sw)
        nbatch = (hi - lo + BT - 1) >> 7

        def batch(it, _):
            base = lo + it * BT
            pltpu.sync_copy(lidxb_hbm.at[pl.ds(b * CAPP + base, BT)], lstage)
            for kk in range(8):
                lv = lstage[pl.ds(kk * 16, 16)]
                valid = (_i16(base + kk * 16) + lane) < _i16(hi)
                lstage_m[pl.ds(kk * 16, 16)] = jnp.where(valid, lv, LR + lane)
            pltpu.sync_copy(ones_rows, cnt_sh.at[lstage_m], add=True)
            return 0

        lax.fori_loop(0, nbatch, batch, 0)
        plsc.subcore_barrier()
        for f4 in range(4):
            pltpu.sync_copy(cnt_sh.at[pl.ds(sw * BN + f4 * 160, 160)], fbuf)
            pltpu.sync_copy(
                fbuf, cnt_hbm.at[pl.ds(b * LR + sw * BN + f4 * 160, 160)])
        plsc.subcore_barrier()


@functools.partial(
    pl.kernel,
    out_type=jax.ShapeDtypeStruct((GR, HID), f32),
    mesh=_mesh,
    compiler_params=pltpu.CompilerParams(needs_layout_passes=False),
    scratch_types=[
        pltpu.VMEM((16,), i32),
        pltpu.VMEM((BT,), i32), pltpu.VMEM((BT,), i32),
        pltpu.VMEM((BT, HID), f32),
        pltpu.VMEM((82, HID), f32), pltpu.VMEM((160, HID), f32),
        pltpu.VMEM_SHARED((ACC_R, HID), f32),
    ],
    name="rgcn_seg_count",
)
def _k_cnt2(lidxb_hbm, tot_hbm, cnt_hbm,
            totv, lstage, lstage_m, ones_rows, zbuf, fbuf, cnt_sh):
    _cnt_body(lidxb_hbm, tot_hbm, cnt_hbm,
              totv, lstage, lstage_m, ones_rows, zbuf, fbuf, cnt_sh)


NBLK = 400


def _dense_body(acc_ref, cnt_ref, h_ref, comp_ref, bases_ref, root_ref,
                bias_ref, o_ref, *, apply_relu):
    inv = 1.0 / jnp.maximum(cnt_ref[...], 1.0)
    out = jnp.dot(h_ref[...], root_ref[...], preferred_element_type=f32)
    bs = [bases_ref[pl.ds(k * HID, HID), :] for k in range(4)]
    for r in range(NRELS):
        wr = (comp_ref[r, 0] * bs[0] + comp_ref[r, 1] * bs[1]
              + comp_ref[r, 2] * bs[2] + comp_ref[r, 3] * bs[3])
        m = acc_ref[:, r, :] * inv[:, r:r + 1]
        out = out + jnp.dot(m, wr, preferred_element_type=f32)
    out = out + bias_ref[...]
    o_ref[...] = jnp.maximum(out, 0.0) if apply_relu else out


def _dense(acc3, cnt2, h, comp, bases2, root, bias2, apply_relu):
    grid = (N // NBLK,)
    return pl.pallas_call(
        functools.partial(_dense_body, apply_relu=apply_relu),
        grid=grid,
        in_specs=[
            pl.BlockSpec((NBLK, NRELS, HID), lambda i: (i, 0, 0)),
            pl.BlockSpec((NBLK, NRELS), lambda i: (i, 0)),
            pl.BlockSpec((NBLK, HID), lambda i: (i, 0)),
            pl.BlockSpec((NRELS, 4), lambda i: (0, 0)),
            pl.BlockSpec((4 * HID, HID), lambda i: (0, 0)),
            pl.BlockSpec((HID, HID), lambda i: (0, 0)),
            pl.BlockSpec((1, HID), lambda i: (0, 0)),
        ],
        out_specs=pl.BlockSpec((NBLK, HID), lambda i: (i, 0)),
        out_shape=jax.ShapeDtypeStruct((N, HID), f32),
        name="rgcn_dense",
    )(acc3, cnt2, h, comp, bases2, root, bias2)


def _head_body(h_hbm, re_hbm, qo_hbm, qr_hbm, zl_hbm, zr_hbm, qiv, rows8, sem):
    wid = _wid()
    base = wid * 8
    pltpu.sync_copy(qo_hbm.at[pl.ds(base, 8)], qiv)
    pltpu.async_copy(h_hbm.at[qiv], rows8, sem).wait()
    pltpu.sync_copy(rows8, zl_hbm.at[pl.ds(base, 8)])
    pltpu.sync_copy(qr_hbm.at[pl.ds(base, 8)], qiv)
    pltpu.async_copy(re_hbm.at[qiv], rows8, sem).wait()
    pltpu.sync_copy(rows8, zr_hbm.at[pl.ds(base, 8)])


@functools.partial(
    pl.kernel,
    out_type=(jax.ShapeDtypeStruct((256, HID), f32),
              jax.ShapeDtypeStruct((256, HID), f32)),
    mesh=_mesh,
    compiler_params=pltpu.CompilerParams(needs_layout_passes=False),
    scratch_types=[
        pltpu.VMEM((8,), i32), pltpu.VMEM((8, HID), f32),
        pltpu.SemaphoreType.DMA,
    ],
    name="rgcn_head_gather",
)
def _k_head(h_hbm, re_hbm, qo_hbm, qr_hbm, zl_hbm, zr_hbm, qiv, rows8, sem):
    _head_body(h_hbm, re_hbm, qo_hbm, qr_hbm, zl_hbm, zr_hbm, qiv, rows8, sem)


def _lin_body(zl_ref, zr_ref, wt_ref, wb_ref, b_ref, o_ref):
    o_ref[...] = (jnp.dot(zl_ref[...], wt_ref[...], preferred_element_type=f32)
                  + jnp.dot(zr_ref[...], wb_ref[...], preferred_element_type=f32)
                  + b_ref[...])


def _final_linear(zl, zr, wt, wb, bp):
    return pl.pallas_call(
        _lin_body,
        out_shape=jax.ShapeDtypeStruct((256, HID), f32),
        name="rgcn_final_linear",
    )(zl, zr, wt, wb, bp)


def kernel(x, node_ent, edge_index, edge_type, dst, ptr, q_rel,
           comp1, bases1, root1, bias1, comp2, bases2, root2, bias2,
           comp3, bases3, root3, bias3, comp4, bases4, root4, bias4,
           rel_emb, lin_w, lin_b):
    grp = jax.nn.one_hot(node_ent, 16, dtype=f32)
    h = jnp.concatenate([x, grp], axis=-1)

    src = edge_index[0].astype(i32)
    dstn = edge_index[1].astype(i32)
    rel = edge_type.astype(i32)

    counts = _k_count(dstn)
    totals = counts.reshape(NW, 16).sum(axis=0).astype(i32)
    srcb, lidxb = _k_bucket(src, dstn, rel, counts)
    cnt_gr = _k_cnt2(lidxb, totals)
    cnt2 = cnt_gr[:N * NRELS, 0].reshape(N, NRELS)

    layers = [(comp1, bases1, root1, bias1, True),
              (comp2, bases2, root2, bias2, True),
              (comp3, bases3, root3, bias3, True),
              (comp4, bases4, root4, bias4, False)]
    for comp, bases, root, bias, relu in layers:
        acc = _k_seg(h, srcb, lidxb, totals)
        acc3 = acc[:N * NRELS].reshape(N, NRELS, HID)
        h = _dense(acc3, cnt2, h, comp, bases.reshape(4 * HID, HID), root,
                   bias.reshape(1, HID), relu)

    qo = (dst + ptr[:-1]).astype(i32)
    zl, zr = _k_head(h, rel_emb, qo, q_rel.astype(i32))

    wp = jnp.zeros((2 * HID, HID), f32).at[:, :2].set(lin_w)
    bp = jnp.zeros((1, HID), f32).at[0, :2].set(lin_b)
    out = _final_linear(zl, zr, wp[:HID], wp[HID:], bp)
    return out[:, :2]


# speedup vs baseline: 1.2967x; 1.2967x over previous
import functools

import jax
import jax.numpy as jnp
from jax import lax
from jax.experimental import pallas as pl
from jax.experimental.pallas import tpu as pltpu
from jax.experimental.pallas import tpu_sc as plsc

f32 = jnp.float32
i32 = jnp.int32

N = 10000
E = 320000
NRELS = 16
HID = 128
NBK = 16
BN = 640
LR = BN * NRELS
ACC_R = 10256
STRIPE = ACC_R // 16
_ZCH = (82, 82, 82, 82, 82, 82, 82, 67)
GR = NBK * LR
CAPP = E + 3072
CV = 11024
NC, NS = 2, 16
NW = NC * NS
EW = E // NW
BT = 128

_mesh = plsc.VectorSubcoreMesh(core_axis_name="c", subcore_axis_name="s")


def _wid():
    return lax.axis_index("s") * NC + lax.axis_index("c")


def _lane():
    return lax.iota(i32, 16)


def _i16(v):
    return jnp.full((16,), v, i32)


_ONE16 = None


def _div640(d):
    return ((d >> 7) * 205) >> 10


def _count16(m):
    return jnp.sum(jnp.where(m, _i16(1), _i16(0)))


def _count_body(dst_hbm, cnt_hbm, dvm, cvm):
    wid = _wid()
    lane = _lane()
    pltpu.sync_copy(dst_hbm.at[pl.ds(wid * EW, EW)], dvm)

    def step(j, accs):
        d = dvm[pl.ds(j * 16, 16)]
        bk = _div640(d)
        return tuple(
            accs[b] + jnp.where(bk == _i16(b), _i16(1), _i16(0))
            for b in range(NBK)
        )

    accs = lax.fori_loop(0, EW // 16, step,
                         tuple(jnp.zeros((16,), i32) for _ in range(NBK)))
    cnts = jnp.zeros((16,), i32)
    for b in range(NBK):
        cnts = jnp.where(lane == _i16(b), _i16(jnp.sum(accs[b])), cnts)
    cvm[...] = cnts
    pltpu.sync_copy(cvm, cnt_hbm.at[pl.ds(wid * 16, 16)])


@functools.partial(
    pl.kernel,
    out_type=jax.ShapeDtypeStruct((NW * 16,), i32),
    mesh=_mesh,
    compiler_params=pltpu.CompilerParams(needs_layout_passes=False),
    scratch_types=[pltpu.VMEM((EW,), i32), pltpu.VMEM((16,), i32)],
    name="rgcn_bucket_count",
)
def _k_count(dst_hbm, cnt_hbm, dvm, cvm):
    _count_body(dst_hbm, cnt_hbm, dvm, cvm)


GV = 5


def _bucket_body(src_hbm, dst_hbm, rel_hbm, cnt_hbm, srcb_hbm, lidxb_hbm,
                 svm, dvm, rvm, cvm, offv, cursv, csrc, clidx):
    wid = _wid()
    lane = _lane()
    pltpu.sync_copy(src_hbm.at[pl.ds(wid * EW, EW)], svm)
    pltpu.sync_copy(dst_hbm.at[pl.ds(wid * EW, EW)], dvm)
    pltpu.sync_copy(rel_hbm.at[pl.ds(wid * EW, EW)], rvm)
    pltpu.sync_copy(cnt_hbm, cvm)

    def r64(x):
        return ((x + _i16(63)) >> 6) << 6

    off = lane * CAPP

    def prior(w, off):
        row = r64(cvm[pl.ds(w * 16, 16)])
        keep = _i16(w) < _i16(wid)
        return off + jnp.where(keep, row, _i16(0))

    off = lax.fori_loop(0, NW, prior, off)
    offv[...] = off

    my64 = r64(cvm[pl.ds(wid * 16, 16)])
    lbase = plsc.cumsum(my64) - my64
    trash_s = lane * 577
    trash_l = _i16(LR) + lane

    def tfill(i, _):
        csrc[pl.ds(i * 16, 16)] = trash_s
        clidx[pl.ds(i * 16, 16)] = trash_l
        return 0

    lax.fori_loop(0, CV // 16, tfill, 0)
    cursv[...] = lbase

    def step(jj, _):
        d = dvm[pl.ds(jj * 16, 16)]
        s_ = svm[pl.ds(jj * 16, 16)]
        r_ = rvm[pl.ds(jj * 16, 16)]
        bk = _div640(d)
        loc = (d - bk * BN) * NRELS + r_
        rank = jnp.zeros((16,), i32)
        later = jnp.zeros((16,), i32)
        for sh in range(1, 16):
            pv = bk.at[(lane - _i16(sh)) & _i16(15)].get(
                mode="promise_in_bounds")
            nx = bk.at[(lane + _i16(sh)) & _i16(15)].get(
                mode="promise_in_bounds")
            rank = rank + jnp.where(
                (lane >= _i16(sh)) & (pv == bk), _i16(1), _i16(0))
            later = later | jnp.where(
                (lane + _i16(sh) < _i16(16)) & (nx == bk), _i16(1), _i16(0))
        pos = plsc.load_gather(cursv, [bk]) + rank
        plsc.store_scatter(csrc, [pos], s_)
        plsc.store_scatter(clidx, [pos], loc)
        plsc.store_scatter(cursv, [bk], pos + _i16(1),
                           mask=later == _i16(0))
        return 0

    lax.fori_loop(0, EW // 16, step, 0)

    for b in range(NBK):
        gof = pl.multiple_of(
            jnp.sum(jnp.where(lane == _i16(b), offv[...], _i16(0))), 64)
        lof = pl.multiple_of(
            jnp.sum(jnp.where(lane == _i16(b), lbase, _i16(0))), 64)
        nf = jnp.sum(jnp.where(lane == _i16(b), my64 >> 6, _i16(0)))

        def flush(i, _):
            pltpu.sync_copy(csrc.at[pl.ds(lof + i * 64, 64)],
                            srcb_hbm.at[pl.ds(gof + i * 64, 64)])
            pltpu.sync_copy(clidx.at[pl.ds(lof + i * 64, 64)],
                            lidxb_hbm.at[pl.ds(gof + i * 64, 64)])
            return 0

        lax.fori_loop(0, nf, flush, 0)


@functools.partial(
    pl.kernel,
    out_type=(jax.ShapeDtypeStruct((NBK * CAPP,), i32),
              jax.ShapeDtypeStruct((NBK * CAPP,), i32)),
    mesh=_mesh,
    compiler_params=pltpu.CompilerParams(needs_layout_passes=False),
    scratch_types=[
        pltpu.VMEM((EW,), i32), pltpu.VMEM((EW,), i32), pltpu.VMEM((EW,), i32),
        pltpu.VMEM((NW * 16,), i32), pltpu.VMEM((16,), i32),
        pltpu.VMEM((16,), i32),
        pltpu.VMEM((CV,), i32), pltpu.VMEM((CV,), i32),
    ],
    name="rgcn_bucket_scatter",
)
def _k_bucket(src_hbm, dst_hbm, rel_hbm, cnt_hbm, srcb_hbm, lidxb_hbm,
              svm, dvm, rvm, cvm, offv, cursv, csrc, clidx):
    _bucket_body(src_hbm, dst_hbm, rel_hbm, cnt_hbm, srcb_hbm, lidxb_hbm,
                 svm, dvm, rvm, cvm, offv, cursv, csrc, clidx)


def _share(totv, b, sw):
    lane = _lane()
    tot = jnp.sum(jnp.where(lane == _i16(b), totv[...], _i16(0)))
    u = (tot + 7) >> 3
    lo = 8 * ((u * sw) >> 4)
    hi = jnp.minimum(8 * ((u * (sw + 1)) >> 4), tot)
    return tot, lo, hi


def _seg_body(h_hbm, srcb_hbm, lidxb_hbm, tot_hbm, acc_hbm,
              totv, sstage, lstage, sA_m, lA_m, sB_m, lB_m, rowsA, rowsB,
              zbuf, acc_sh, semA, semB):
    c = lax.axis_index("c")
    sw = lax.axis_index("s")
    lane = _lane()
    pltpu.sync_copy(tot_hbm, totv)
    zv = jnp.zeros((16,), f32)
    bufs = ((sA_m, lA_m, rowsA, semA), (sB_m, lB_m, rowsB, semB))

    def zfill(i, _):
        for kk in range(8):
            zbuf[i, pl.ds(kk * 16, 16)] = zv
        return 0

    lax.fori_loop(0, 82, zfill, 0)

    for bl in range(NBK // NC):
        b = c * (NBK // NC) + bl
        zo = 0
        for zch in _ZCH:
            pltpu.sync_copy(zbuf.at[pl.ds(0, zch)],
                            acc_sh.at[pl.ds(sw * STRIPE + zo, zch)])
            zo += zch
        plsc.subcore_barrier()
        _, lo, hi = _share(totv, b, sw)
        nbatch = (hi - lo + BT - 1) >> 7

        def prep(it, par):
            base = lo + it * BT
            sm, lm, rw, se = bufs[par]
            pltpu.sync_copy(srcb_hbm.at[pl.ds(b * CAPP + base, BT)], sstage)
            pltpu.sync_copy(lidxb_hbm.at[pl.ds(b * CAPP + base, BT)], lstage)
            for kk in range(8):
                sv = sstage[pl.ds(kk * 16, 16)]
                lv = lstage[pl.ds(kk * 16, 16)]
                valid = (_i16(base + kk * 16) + lane) < _i16(hi)
                sm[pl.ds(kk * 16, 16)] = jnp.where(valid, sv, lane * 577)
                lm[pl.ds(kk * 16, 16)] = jnp.where(valid, lv, LR + lane)
            pltpu.async_copy(h_hbm.at[sm], rw, se)

        @pl.when(nbatch > 0)
        def _():
            prep(0, 0)

        def pipe(i2, _):
            for par in range(2):
                it = i2 * 2 + par

                @pl.when(it + 1 < nbatch)
                def _():
                    prep(it + 1, 1 - par)

                @pl.when(it < nbatch)
                def _():
                    sm, lm, rw, se = bufs[par]
                    pltpu.make_async_copy(h_hbm.at[sm], rw, se).wait()
                    pltpu.sync_copy(rw, acc_sh.at[lm], add=True)
            return 0

        lax.fori_loop(0, (nbatch + 1) >> 1, pipe, 0)
        plsc.subcore_barrier()
        pltpu.sync_copy(acc_sh.at[pl.ds(sw * BN, BN)],
                        acc_hbm.at[pl.ds(b * LR + sw * BN, BN)])
        plsc.subcore_barrier()


@functools.partial(
    pl.kernel,
    out_type=jax.ShapeDtypeStruct((GR, HID), f32),
    mesh=_mesh,
    compiler_params=pltpu.CompilerParams(needs_layout_passes=False),
    scratch_types=[
        pltpu.VMEM((16,), i32),
        pltpu.VMEM((BT,), i32), pltpu.VMEM((BT,), i32),
        pltpu.VMEM((BT,), i32), pltpu.VMEM((BT,), i32),
        pltpu.VMEM((BT,), i32), pltpu.VMEM((BT,), i32),
        pltpu.VMEM((BT, HID), f32), pltpu.VMEM((BT, HID), f32),
        pltpu.VMEM((82, HID), f32),
        pltpu.VMEM_SHARED((ACC_R, HID), f32),
        pltpu.SemaphoreType.DMA, pltpu.SemaphoreType.DMA,
    ],
    name="rgcn_seg_sum",
)
def _k_seg(h_hbm, srcb_hbm, lidxb_hbm, tot_hbm, acc_hbm,
           totv, sstage, lstage, sA_m, lA_m, sB_m, lB_m, rowsA, rowsB,
           zbuf, acc_sh, semA, semB):
    _seg_body(h_hbm, srcb_hbm, lidxb_hbm, tot_hbm, acc_hbm,
              totv, sstage, lstage, sA_m, lA_m, sB_m, lB_m, rowsA, rowsB,
              zbuf, acc_sh, semA, semB)


def _cnt_body(lidxb_hbm, tot_hbm, cnt_hbm,
              totv, lstage, lstage_m, ones_rows, zbuf, fbuf, cnt_sh):
    c = lax.axis_index("c")
    sw = lax.axis_index("s")
    lane = _lane()
    pltpu.sync_copy(tot_hbm, totv)
    zv = jnp.zeros((16,), f32)
    ov = jnp.full((16,), 1.0, f32)

    def zfill(i, _):
        for kk in range(8):
            zbuf[i, pl.ds(kk * 16, 16)] = zv
            ones_rows[i, pl.ds(kk * 16, 16)] = ov
        return 0

    lax.fori_loop(0, 82, zfill, 0)

    def ofill(i, _):
        for kk in range(8):
            ones_rows[82 + i, pl.ds(kk * 16, 16)] = ov
        return 0

    lax.fori_loop(0, BT - 82, ofill, 0)

    for bl in range(NBK // NC):
        b = c * (NBK // NC) + bl
        zo = 0
        for zch in _ZCH:
            pltpu.sync_copy(zbuf.at[pl.ds(0, zch)],
                            cnt_sh.at[pl.ds(sw * STRIPE + zo, zch)])
            zo += zch
        plsc.subcore_barrier()
        _, lo, hi = _share(totv, b, sw)
        nbatch = (hi - lo + BT - 1) >> 7

        def batch(it, _):
            base = lo + it * BT
            pltpu.sync_copy(lidxb_hbm.at[pl.ds(b * CAPP + base, BT)], lstage)
            for kk in range(8):
                lv = lstage[pl.ds(kk * 16, 16)]
                valid = (_i16(base + kk * 16) + lane) < _i16(hi)
                lstage_m[pl.ds(kk * 16, 16)] = jnp.where(valid, lv, LR + lane)
            pltpu.sync_copy(ones_rows, cnt_sh.at[lstage_m], add=True)
            return 0

        lax.fori_loop(0, nbatch, batch, 0)
        plsc.subcore_barrier()
        for f4 in range(4):
            pltpu.sync_copy(cnt_sh.at[pl.ds(sw * BN + f4 * 160, 160)], fbuf)
            pltpu.sync_copy(
                fbuf, cnt_hbm.at[pl.ds(b * LR + sw * BN + f4 * 160, 160)])
        plsc.subcore_barrier()


@functools.partial(
    pl.kernel,
    out_type=jax.ShapeDtypeStruct((GR, HID), f32),
    mesh=_mesh,
    compiler_params=pltpu.CompilerParams(needs_layout_passes=False),
    scratch_types=[
        pltpu.VMEM((16,), i32),
        pltpu.VMEM((BT,), i32), pltpu.VMEM((BT,), i32),
        pltpu.VMEM((BT, HID), f32),
        pltpu.VMEM((82, HID), f32), pltpu.VMEM((160, HID), f32),
        pltpu.VMEM_SHARED((ACC_R, HID), f32),
    ],
    name="rgcn_seg_count",
)
def _k_cnt2(lidxb_hbm, tot_hbm, cnt_hbm,
            totv, lstage, lstage_m, ones_rows, zbuf, fbuf, cnt_sh):
    _cnt_body(lidxb_hbm, tot_hbm, cnt_hbm,
              totv, lstage, lstage_m, ones_rows, zbuf, fbuf, cnt_sh)


NBLK = 400


def _dense_body(acc_ref, cnt_ref, h_ref, comp_ref, bases_ref, root_ref,
                bias_ref, o_ref, *, apply_relu):
    inv = 1.0 / jnp.maximum(cnt_ref[...], 1.0)
    out = jnp.dot(h_ref[...], root_ref[...], preferred_element_type=f32)
    bs = [bases_ref[pl.ds(k * HID, HID), :] for k in range(4)]
    for r in range(NRELS):
        wr = (comp_ref[r, 0] * bs[0] + comp_ref[r, 1] * bs[1]
              + comp_ref[r, 2] * bs[2] + comp_ref[r, 3] * bs[3])
        m = acc_ref[:, r, :] * inv[:, r:r + 1]
        out = out + jnp.dot(m, wr, preferred_element_type=f32)
    out = out + bias_ref[...]
    o_ref[...] = jnp.maximum(out, 0.0) if apply_relu else out


def _dense(acc3, cnt2, h, comp, bases2, root, bias2, apply_relu):
    grid = (N // NBLK,)
    return pl.pallas_call(
        functools.partial(_dense_body, apply_relu=apply_relu),
        grid=grid,
        in_specs=[
            pl.BlockSpec((NBLK, NRELS, HID), lambda i: (i, 0, 0)),
            pl.BlockSpec((NBLK, NRELS), lambda i: (i, 0)),
            pl.BlockSpec((NBLK, HID), lambda i: (i, 0)),
            pl.BlockSpec((NRELS, 4), lambda i: (0, 0)),
            pl.BlockSpec((4 * HID, HID), lambda i: (0, 0)),
            pl.BlockSpec((HID, HID), lambda i: (0, 0)),
            pl.BlockSpec((1, HID), lambda i: (0, 0)),
        ],
        out_specs=pl.BlockSpec((NBLK, HID), lambda i: (i, 0)),
        out_shape=jax.ShapeDtypeStruct((N, HID), f32),
        name="rgcn_dense",
    )(acc3, cnt2, h, comp, bases2, root, bias2)


def _head_body(h_hbm, re_hbm, qo_hbm, qr_hbm, zl_hbm, zr_hbm, qiv, rows8, sem):
    wid = _wid()
    base = wid * 8
    pltpu.sync_copy(qo_hbm.at[pl.ds(base, 8)], qiv)
    pltpu.async_copy(h_hbm.at[qiv], rows8, sem).wait()
    pltpu.sync_copy(rows8, zl_hbm.at[pl.ds(base, 8)])
    pltpu.sync_copy(qr_hbm.at[pl.ds(base, 8)], qiv)
    pltpu.async_copy(re_hbm.at[qiv], rows8, sem).wait()
    pltpu.sync_copy(rows8, zr_hbm.at[pl.ds(base, 8)])


@functools.partial(
    pl.kernel,
    out_type=(jax.ShapeDtypeStruct((256, HID), f32),
              jax.ShapeDtypeStruct((256, HID), f32)),
    mesh=_mesh,
    compiler_params=pltpu.CompilerParams(needs_layout_passes=False),
    scratch_types=[
        pltpu.VMEM((8,), i32), pltpu.VMEM((8, HID), f32),
        pltpu.SemaphoreType.DMA,
    ],
    name="rgcn_head_gather",
)
def _k_head(h_hbm, re_hbm, qo_hbm, qr_hbm, zl_hbm, zr_hbm, qiv, rows8, sem):
    _head_body(h_hbm, re_hbm, qo_hbm, qr_hbm, zl_hbm, zr_hbm, qiv, rows8, sem)


def _lin_body(zl_ref, zr_ref, wt_ref, wb_ref, b_ref, o_ref):
    o_ref[...] = (jnp.dot(zl_ref[...], wt_ref[...], preferred_element_type=f32)
                  + jnp.dot(zr_ref[...], wb_ref[...], preferred_element_type=f32)
                  + b_ref[...])


def _final_linear(zl, zr, wt, wb, bp):
    return pl.pallas_call(
        _lin_body,
        out_shape=jax.ShapeDtypeStruct((256, HID), f32),
        name="rgcn_final_linear",
    )(zl, zr, wt, wb, bp)


def kernel(x, node_ent, edge_index, edge_type, dst, ptr, q_rel,
           comp1, bases1, root1, bias1, comp2, bases2, root2, bias2,
           comp3, bases3, root3, bias3, comp4, bases4, root4, bias4,
           rel_emb, lin_w, lin_b):
    grp = jax.nn.one_hot(node_ent, 16, dtype=f32)
    h = jnp.concatenate([x, grp], axis=-1)

    src = edge_index[0].astype(i32)
    dstn = edge_index[1].astype(i32)
    rel = edge_type.astype(i32)

    counts = _k_count(dstn)
    totals = (((counts.reshape(NW, 16) + 63) // 64) * 64).sum(axis=0).astype(i32)
    srcb, lidxb = _k_bucket(src, dstn, rel, counts)
    cnt_gr = _k_cnt2(lidxb, totals)
    cnt2 = cnt_gr[:N * NRELS, 0].reshape(N, NRELS)

    layers = [(comp1, bases1, root1, bias1, True),
              (comp2, bases2, root2, bias2, True),
              (comp3, bases3, root3, bias3, True),
              (comp4, bases4, root4, bias4, False)]
    for comp, bases, root, bias, relu in layers:
        acc = _k_seg(h, srcb, lidxb, totals)
        acc3 = acc[:N * NRELS].reshape(N, NRELS, HID)
        h = _dense(acc3, cnt2, h, comp, bases.reshape(4 * HID, HID), root,
                   bias.reshape(1, HID), relu)

    qo = (dst + ptr[:-1]).astype(i32)
    zl, zr = _k_head(h, rel_emb, qo, q_rel.astype(i32))

    wp = jnp.zeros((2 * HID, HID), f32).at[:, :2].set(lin_w)
    bp = jnp.zeros((1, HID), f32).at[0, :2].set(lin_b)
    out = _final_linear(zl, zr, wp[:HID], wp[HID:], bp)
    return out[:, :2]
